# baseline (speedup 1.0000x reference)
import functools

import jax
import jax.numpy as jnp
from jax.experimental import pallas as pl
from jax.experimental.pallas import tpu as pltpu
from jax.experimental.pallas import tpu_sc as plsc

_N = 5000
_THR = 0.6
_B = 1024
_NP = 5120
_NB = _NP // _B

_NW = 32
_BPW = _NP // _NW
_CH = _BPW // 2


def _sc_gather_body(tbl_hbm, idx_hbm, out_hbm, idx_v, rows_v, sem):
    wid = jax.lax.axis_index("s") * 2 + jax.lax.axis_index("c")
    for c in range(_BPW // _CH):
        base = wid * _BPW + c * _CH
        pltpu.sync_copy(idx_hbm.at[pl.ds(base, _CH)], idx_v)
        pltpu.async_copy(tbl_hbm.at[idx_v], rows_v, sem).wait()
        pltpu.sync_copy(rows_v, out_hbm.at[pl.ds(base, _CH)])


_sc_gather = functools.partial(
    pl.kernel,
    mesh=plsc.VectorSubcoreMesh(core_axis_name="c", subcore_axis_name="s"),
    out_type=jax.ShapeDtypeStruct((_NP, 16), jnp.float32),
    scratch_types=[
        pltpu.VMEM((_CH,), jnp.int32),
        pltpu.VMEM((_CH, 16), jnp.float32),
        pltpu.SemaphoreType.DMA,
    ],
    compiler_params=pltpu.CompilerParams(use_tc_tiling_on_sc=False),
)(_sc_gather_body)


def _nms_body(cint_ref, out_ref, keep_ref, cin_ref):
    keep_ref[...] = jnp.ones((1, _NP), jnp.float32)
    cin_ref[...] = cint_ref[...].T

    rid = jax.lax.broadcasted_iota(jnp.int32, (_B, _B), 0)
    cid = jax.lax.broadcasted_iota(jnp.int32, (_B, _B), 1)
    upper = (cid > rid).astype(jnp.bfloat16)

    def tile_sup(i, m):
        br = cint_ref[pl.ds(i * _B, _B), :]
        bc = cin_ref[:, pl.ds(m * _B, _B)]
        x1r, y1r, x2r, y2r = br[:, 0:1], br[:, 1:2], br[:, 2:3], br[:, 3:4]
        x1c, y1c, x2c, y2c = bc[0:1, :], bc[1:2, :], bc[2:3, :], bc[3:4, :]
        ar = jnp.maximum(x2r - x1r, 0.0) * jnp.maximum(y2r - y1r, 0.0)
        ac = jnp.maximum(x2c - x1c, 0.0) * jnp.maximum(y2c - y1c, 0.0)
        xx1 = jnp.maximum(x1r, x1c)
        yy1 = jnp.maximum(y1r, y1c)
        xx2 = jnp.minimum(x2r, x2c)
        yy2 = jnp.minimum(y2r, y2c)
        inter = jnp.maximum(xx2 - xx1, 0.0) * jnp.maximum(yy2 - yy1, 0.0)
        union = ar + ac - inter
        iou = inter / union
        return (iou > _THR).astype(jnp.bfloat16)

    def matvec(kb, sup):
        return jax.lax.dot_general(
            kb, sup, (((1,), (0,)), ((), ())),
            preferred_element_type=jnp.float32)

    for i in range(_NB):
        supd = tile_sup(i, i) * upper
        inc = keep_ref[:, pl.ds(i * _B, _B)]

        def fp_cond(c):
            return c[1] > 0

        def fp_body(c, supd=supd, inc=inc):
            kb, _ = c
            cnt = matvec(kb.astype(jnp.bfloat16), supd)
            new = inc * (cnt == 0.0).astype(jnp.float32)
            changed = jnp.any(new != kb).astype(jnp.int32)
            return (new, changed)

        kb, _ = jax.lax.while_loop(fp_cond, fp_body, (inc, jnp.int32(1)))
        keep_ref[:, pl.ds(i * _B, _B)] = kb
        kbh = kb.astype(jnp.bfloat16)

        for m in range(i + 1, _NB):
            cnt = matvec(kbh, tile_sup(i, m))
            kr = keep_ref[:, pl.ds(m * _B, _B)]
            keep_ref[:, pl.ds(m * _B, _B)] = kr * (cnt == 0.0).astype(
                jnp.float32)

    kcol = keep_ref[...].T
    out_ref[:, 0:4] = cint_ref[:, 0:4] * kcol
    out_ref[:, 4:5] = jnp.sqrt(jnp.maximum(cint_ref[:, 4:5], 1e-8)) * kcol


def kernel(boxes, scores):
    _, order = jax.lax.sort_key_val(
        -scores, jnp.arange(_N, dtype=jnp.int32))
    tbl16 = jnp.pad(
        jnp.concatenate([boxes, scores[:, None]], axis=1),
        ((0, 0), (0, 11)))
    orderp = jnp.pad(order, (0, _NP - _N))
    cint = _sc_gather(tbl16, orderp)
    outp = pl.pallas_call(
        _nms_body,
        out_shape=jax.ShapeDtypeStruct((_NP, 8), jnp.float32),
        scratch_shapes=[
            pltpu.VMEM((1, _NP), jnp.float32),
            pltpu.VMEM((16, _NP), jnp.float32),
        ],
    )(cint)
    return outp[:_N, :5]


# speedup vs baseline: 1.0134x; 1.0134x over previous
import functools

import jax
import jax.numpy as jnp
from jax.experimental import pallas as pl
from jax.experimental.pallas import tpu as pltpu
from jax.experimental.pallas import tpu_sc as plsc

_N = 5000
_THR = 0.6
_B = 1024
_NP = 5120
_NB = _NP // _B

_NW = 32
_BPW = _NP // _NW
_CH = _BPW // 2


def _sc_gather_body(tbl_hbm, idx_hbm, out_hbm, idx_v, rows_v, sem):
    wid = jax.lax.axis_index("s") * 2 + jax.lax.axis_index("c")
    for c in range(_BPW // _CH):
        base = wid * _BPW + c * _CH
        pltpu.sync_copy(idx_hbm.at[pl.ds(base, _CH)], idx_v)
        pltpu.async_copy(tbl_hbm.at[idx_v], rows_v, sem).wait()
        pltpu.sync_copy(rows_v, out_hbm.at[pl.ds(base, _CH)])


_sc_gather = functools.partial(
    pl.kernel,
    mesh=plsc.VectorSubcoreMesh(core_axis_name="c", subcore_axis_name="s"),
    out_type=jax.ShapeDtypeStruct((_NP, 16), jnp.float32),
    scratch_types=[
        pltpu.VMEM((_CH,), jnp.int32),
        pltpu.VMEM((_CH, 16), jnp.float32),
        pltpu.SemaphoreType.DMA,
    ],
    compiler_params=pltpu.CompilerParams(use_tc_tiling_on_sc=False),
)(_sc_gather_body)


def _nms_body(cint_ref, out_ref, keep_ref, cin_ref):
    keep_ref[...] = jnp.ones((1, _NP), jnp.float32)
    cin_ref[...] = cint_ref[...].T

    rid = jax.lax.broadcasted_iota(jnp.int32, (_B, _B), 0)
    cid = jax.lax.broadcasted_iota(jnp.int32, (_B, _B), 1)
    upper = (cid > rid).astype(jnp.float32)

    def tile_sup(i, m):
        br = cint_ref[pl.ds(i * _B, _B), :]
        bc = cin_ref[:, pl.ds(m * _B, _B)]
        x1r, y1r, x2r, y2r = br[:, 0:1], br[:, 1:2], br[:, 2:3], br[:, 3:4]
        x1c, y1c, x2c, y2c = bc[0:1, :], bc[1:2, :], bc[2:3, :], bc[3:4, :]
        ar = jnp.maximum(x2r - x1r, 0.0) * jnp.maximum(y2r - y1r, 0.0)
        ac = jnp.maximum(x2c - x1c, 0.0) * jnp.maximum(y2c - y1c, 0.0)
        xx1 = jnp.maximum(x1r, x1c)
        yy1 = jnp.maximum(y1r, y1c)
        xx2 = jnp.minimum(x2r, x2c)
        yy2 = jnp.minimum(y2r, y2c)
        inter = jnp.maximum(xx2 - xx1, 0.0) * jnp.maximum(yy2 - yy1, 0.0)
        union = ar + ac - inter
        iou = inter / union
        return (iou > _THR).astype(jnp.float32)

    def matvec(kb, sup):
        return jax.lax.dot_general(
            kb, sup, (((1,), (0,)), ((), ())),
            preferred_element_type=jnp.float32)

    for i in range(_NB):
        supd = tile_sup(i, i) * upper
        inc = keep_ref[:, pl.ds(i * _B, _B)]

        def fp_cond(c):
            return c[1] > 0

        def fp_body(c, supd=supd, inc=inc):
            kb, _ = c
            cnt = matvec(kb, supd)
            new = inc * (cnt == 0.0).astype(jnp.float32)
            changed = jnp.any(new != kb).astype(jnp.int32)
            return (new, changed)

        kb, _ = jax.lax.while_loop(fp_cond, fp_body, (inc, jnp.int32(1)))
        keep_ref[:, pl.ds(i * _B, _B)] = kb

        for m in range(i + 1, _NB):
            cnt = matvec(kb, tile_sup(i, m))
            kr = keep_ref[:, pl.ds(m * _B, _B)]
            keep_ref[:, pl.ds(m * _B, _B)] = kr * (cnt == 0.0).astype(
                jnp.float32)

    kcol = keep_ref[...].T
    out_ref[:, 0:4] = cint_ref[:, 0:4] * kcol
    out_ref[:, 4:5] = jnp.sqrt(jnp.maximum(cint_ref[:, 4:5], 1e-8)) * kcol


def kernel(boxes, scores):
    _, order = jax.lax.sort_key_val(
        -scores, jnp.arange(_N, dtype=jnp.int32))
    tbl16 = jnp.pad(
        jnp.concatenate([boxes, scores[:, None]], axis=1),
        ((0, 0), (0, 11)))
    orderp = jnp.pad(order, (0, _NP - _N))
    cint = _sc_gather(tbl16, orderp)
    outp = pl.pallas_call(
        _nms_body,
        out_shape=jax.ShapeDtypeStruct((_NP, 8), jnp.float32),
        scratch_shapes=[
            pltpu.VMEM((1, _NP), jnp.float32),
            pltpu.VMEM((16, _NP), jnp.float32),
        ],
    )(cint)
    return outp[:_N, :5]


# speedup vs baseline: 1.0705x; 1.0564x over previous
import functools

import jax
import jax.numpy as jnp
from jax.experimental import pallas as pl
from jax.experimental.pallas import tpu as pltpu
from jax.experimental.pallas import tpu_sc as plsc

_N = 5000
_THR = 0.6
_B = 1024
_NP = 5120
_NB = _NP // _B

_NW = 32
_BPW = _NP // _NW
_CH = _BPW // 2


def _sc_gather_body(tbl_hbm, idx_hbm, out_hbm, idx_v, rows_v, sem):
    wid = jax.lax.axis_index("s") * 2 + jax.lax.axis_index("c")
    for c in range(_BPW // _CH):
        base = wid * _BPW + c * _CH
        pltpu.sync_copy(idx_hbm.at[pl.ds(base, _CH)], idx_v)
        pltpu.async_copy(tbl_hbm.at[idx_v], rows_v, sem).wait()
        pltpu.sync_copy(rows_v, out_hbm.at[pl.ds(base, _CH)])


_sc_gather = functools.partial(
    pl.kernel,
    mesh=plsc.VectorSubcoreMesh(core_axis_name="c", subcore_axis_name="s"),
    out_type=jax.ShapeDtypeStruct((_NP, 16), jnp.float32),
    scratch_types=[
        pltpu.VMEM((_CH,), jnp.int32),
        pltpu.VMEM((_CH, 16), jnp.float32),
        pltpu.SemaphoreType.DMA,
    ],
    compiler_params=pltpu.CompilerParams(use_tc_tiling_on_sc=False),
)(_sc_gather_body)


def _nms_body(cint_ref, out_ref, keep_ref, cin_ref):
    keep_ref[...] = jnp.ones((1, _NP), jnp.float32)
    cin_ref[...] = cint_ref[...].T

    rid = jax.lax.broadcasted_iota(jnp.int32, (_B, _B), 0)
    cid = jax.lax.broadcasted_iota(jnp.int32, (_B, _B), 1)
    upper = (cid > rid).astype(jnp.float32)

    def tile_sup(i, m):
        br = cint_ref[pl.ds(i * _B, _B), :]
        bc = cin_ref[:, pl.ds(m * _B, _B)]
        x1r, y1r, x2r, y2r = br[:, 0:1], br[:, 1:2], br[:, 2:3], br[:, 3:4]
        x1c, y1c, x2c, y2c = bc[0:1, :], bc[1:2, :], bc[2:3, :], bc[3:4, :]
        ar = jnp.maximum(x2r - x1r, 0.0) * jnp.maximum(y2r - y1r, 0.0)
        ac = jnp.maximum(x2c - x1c, 0.0) * jnp.maximum(y2c - y1c, 0.0)
        xx1 = jnp.maximum(x1r, x1c)
        yy1 = jnp.maximum(y1r, y1c)
        xx2 = jnp.minimum(x2r, x2c)
        yy2 = jnp.minimum(y2r, y2c)
        inter = jnp.maximum(xx2 - xx1, 0.0) * jnp.maximum(yy2 - yy1, 0.0)
        union = ar + ac - inter
        iou = inter / union
        return (iou > _THR).astype(jnp.float32)

    def matvec(kb, sup):
        return jax.lax.dot_general(
            kb, sup, (((1,), (0,)), ((), ())),
            preferred_element_type=jnp.float32)

    for i in range(_NB):
        supd = tile_sup(i, i) * upper
        inc = keep_ref[:, pl.ds(i * _B, _B)]

        def fp_cond(c):
            return c[1] > 0

        def fp_body(c, supd=supd, inc=inc):
            kb, _ = c
            cnt = matvec(kb, supd)
            new = inc * (cnt == 0.0).astype(jnp.float32)
            changed = jnp.any(new != kb).astype(jnp.int32)
            return (new, changed)

        kb, _ = jax.lax.while_loop(fp_cond, fp_body, (inc, jnp.int32(1)))
        keep_ref[:, pl.ds(i * _B, _B)] = kb

        for m in range(i + 1, _NB):
            cnt = matvec(kb, tile_sup(i, m))
            kr = keep_ref[:, pl.ds(m * _B, _B)]
            keep_ref[:, pl.ds(m * _B, _B)] = kr * (cnt == 0.0).astype(
                jnp.float32)

    k = keep_ref[...]
    out_ref[0:4, :] = cin_ref[0:4, :] * k
    out_ref[4:5, :] = jnp.sqrt(jnp.maximum(cin_ref[4:5, :], 1e-8)) * k
    out_ref[5:8, :] = jnp.zeros((3, _NP), jnp.float32)


def kernel(boxes, scores):
    _, order = jax.lax.sort_key_val(
        -scores, jnp.arange(_N, dtype=jnp.int32))
    tbl16 = jnp.pad(
        jnp.concatenate([boxes, scores[:, None]], axis=1),
        ((0, 0), (0, 11)))
    orderp = jnp.pad(order, (0, _NP - _N))
    cint = _sc_gather(tbl16, orderp)
    out8 = pl.pallas_call(
        _nms_body,
        out_shape=jax.ShapeDtypeStruct((8, _NP), jnp.float32),
        scratch_shapes=[
            pltpu.VMEM((1, _NP), jnp.float32),
            pltpu.VMEM((16, _NP), jnp.float32),
        ],
    )(cint)
    return out8[:5, :_N].T


# speedup vs baseline: 1.1585x; 1.0822x over previous
import functools

import jax
import jax.numpy as jnp
from jax.experimental import pallas as pl
from jax.experimental.pallas import tpu as pltpu
from jax.experimental.pallas import tpu_sc as plsc

_N = 5000
_THR = 0.6
_B = 1024
_NP = 5120
_NB = _NP // _B

_NW = 32
_BPW = _NP // _NW
_CH = _BPW // 2


def _sc_gather_body(tbl_hbm, idx_hbm, out_hbm, idx_v, rows_v, sem):
    wid = jax.lax.axis_index("s") * 2 + jax.lax.axis_index("c")
    for c in range(_BPW // _CH):
        base = wid * _BPW + c * _CH
        pltpu.sync_copy(idx_hbm.at[pl.ds(base, _CH)], idx_v)
        pltpu.async_copy(tbl_hbm.at[idx_v], rows_v, sem).wait()
        pltpu.sync_copy(rows_v, out_hbm.at[pl.ds(base, _CH)])


_sc_gather = functools.partial(
    pl.kernel,
    mesh=plsc.VectorSubcoreMesh(core_axis_name="c", subcore_axis_name="s"),
    out_type=jax.ShapeDtypeStruct((_NP, 16), jnp.float32),
    scratch_types=[
        pltpu.VMEM((_CH,), jnp.int32),
        pltpu.VMEM((_CH, 16), jnp.float32),
        pltpu.SemaphoreType.DMA,
    ],
    compiler_params=pltpu.CompilerParams(use_tc_tiling_on_sc=False),
)(_sc_gather_body)


def _nms_body(cint_ref, out_ref, keep_ref, cin_ref):
    keep_ref[...] = jnp.ones((1, _NP), jnp.float32)
    cin_ref[...] = cint_ref[...].T

    rid = jax.lax.broadcasted_iota(jnp.int32, (_B, _B), 0)
    cid = jax.lax.broadcasted_iota(jnp.int32, (_B, _B), 1)
    upper = cid > rid

    def tile_sup(i, m, mask=None):
        br = cint_ref[pl.ds(i * _B, _B), :]
        bc = cin_ref[:, pl.ds(m * _B, _B)]
        x1r, y1r, x2r, y2r = br[:, 0:1], br[:, 1:2], br[:, 2:3], br[:, 3:4]
        x1c, y1c, x2c, y2c = bc[0:1, :], bc[1:2, :], bc[2:3, :], bc[3:4, :]
        ar = jnp.maximum(x2r - x1r, 0.0) * jnp.maximum(y2r - y1r, 0.0)
        ac = jnp.maximum(x2c - x1c, 0.0) * jnp.maximum(y2c - y1c, 0.0)
        xx1 = jnp.maximum(x1r, x1c)
        yy1 = jnp.maximum(y1r, y1c)
        xx2 = jnp.minimum(x2r, x2c)
        yy2 = jnp.minimum(y2r, y2c)
        inter = jnp.maximum(xx2 - xx1, 0.0) * jnp.maximum(yy2 - yy1, 0.0)
        union = ar + ac - inter
        iou = inter / union
        sup = iou > _THR
        if mask is not None:
            sup = sup & mask
        return sup.astype(jnp.float32)

    def matvec(kb, sup):
        return jax.lax.dot_general(
            kb, sup, (((1,), (0,)), ((), ())),
            preferred_element_type=jnp.float32)

    for i in range(_NB):
        supd = tile_sup(i, i, mask=upper)
        inc = keep_ref[:, pl.ds(i * _B, _B)]

        def fp_cond(c):
            return c[1] > 0

        def fp_body(c, supd=supd, inc=inc):
            kb, _ = c
            cnt = matvec(kb, supd)
            new = inc * (cnt == 0.0).astype(jnp.float32)
            changed = jnp.any(new != kb).astype(jnp.int32)
            return (new, changed)

        kb, _ = jax.lax.while_loop(fp_cond, fp_body, (inc, jnp.int32(1)))
        keep_ref[:, pl.ds(i * _B, _B)] = kb

        for m in range(i + 1, _NB):
            cnt = matvec(kb, tile_sup(i, m))
            kr = keep_ref[:, pl.ds(m * _B, _B)]
            keep_ref[:, pl.ds(m * _B, _B)] = kr * (cnt == 0.0).astype(
                jnp.float32)

    k = keep_ref[...]
    out_ref[0:4, :] = cin_ref[0:4, :] * k
    out_ref[4:5, :] = jnp.sqrt(jnp.maximum(cin_ref[4:5, :], 1e-8)) * k
    out_ref[5:8, :] = jnp.zeros((3, _NP), jnp.float32)


def kernel(boxes, scores):
    _, order = jax.lax.sort_key_val(
        -scores, jnp.arange(_N, dtype=jnp.int32))
    tbl16 = jnp.pad(
        jnp.concatenate([boxes, scores[:, None]], axis=1),
        ((0, 0), (0, 11)))
    orderp = jnp.pad(order, (0, _NP - _N))
    cint = _sc_gather(tbl16, orderp)
    out8 = pl.pallas_call(
        _nms_body,
        out_shape=jax.ShapeDtypeStruct((8, _NP), jnp.float32),
        scratch_shapes=[
            pltpu.VMEM((1, _NP), jnp.float32),
            pltpu.VMEM((16, _NP), jnp.float32),
        ],
    )(cint)
    return out8[:5, :_N].T
